# baseline (speedup 1.0000x reference)
import functools

import numpy as np
import jax
import jax.numpy as jnp
from jax import lax
from jax.experimental import pallas as pl
from jax.experimental.pallas import tpu as pltpu
from jax.experimental.pallas import tpu_sc as plsc

_SEQ = 8192
_D = 512
_LANES = 16
_NC = 2
_NS = 16
_NW = _NC * _NS
_BPW = _SEQ // _NW
_C = 32
_NCH = _BPW // _C
_NBUF = 4
_PBUF = 2


def _positional_encodings_np(seq_len, d):
    pos = np.arange(seq_len, dtype=np.float64)[:, None]
    hid = np.arange(d, dtype=np.float64)[None, :]
    angles = pos / np.power(10000.0, 2.0 * (np.floor(hid / 2.0)) / d)
    pe = np.array(angles)
    pe[:, 0::2] = np.sin(angles[:, 0::2])
    pe[:, 1::2] = np.cos(angles[:, 1::2])
    return pe.astype(np.float32)


_PE_NP = _positional_encodings_np(_SEQ, _D)


def _body(x3_hbm, table_hbm, pe_hbm, out_hbm, idx_v, rows_v, pe_v,
          sem_g, sem_p, sem_o):
    wid = lax.axis_index("s") * _NC + lax.axis_index("c")
    base = wid * _BPW
    pltpu.sync_copy(x3_hbm.at[wid], idx_v)

    def start_gather(ch):
        return pltpu.async_copy(
            table_hbm.at[idx_v.at[ch]], rows_v.at[ch % _NBUF],
            sem_g.at[ch % _NBUF])

    def start_pe(ch):
        return pltpu.async_copy(
            pe_hbm.at[pl.ds(base + ch * _C, _C)], pe_v.at[ch % _PBUF],
            sem_p.at[ch % _PBUF])

    g, p, o = {}, {}, {}
    for ch in range(_PBUF):
        g[ch] = start_gather(ch)
        p[ch] = start_pe(ch)

    for ch in range(_NCH):
        b = ch % _NBUF
        pb = ch % _PBUF
        nxt = ch + _PBUF
        if nxt < _NCH:
            if nxt - _NBUF >= 0:
                o[nxt - _NBUF].wait()
            g[nxt] = start_gather(nxt)
        g[ch].wait()
        p[ch].wait()

        @plsc.parallel_loop(0, _C, step=1, unroll=4)
        def _add(r):
            for j in range(_D // _LANES):
                off = j * _LANES
                rows_v[b, r, pl.ds(off, _LANES)] = (
                    rows_v[b, r, pl.ds(off, _LANES)]
                    + pe_v[pb, r, pl.ds(off, _LANES)]
                )

        o[ch] = pltpu.async_copy(
            rows_v.at[b], out_hbm.at[pl.ds(base + ch * _C, _C)], sem_o.at[b])
        if nxt < _NCH:
            p[nxt] = start_pe(nxt)

    for ch in range(_NCH - min(_NBUF, _NCH), _NCH):
        o[ch].wait()


_sc_kernel = functools.partial(
    pl.kernel,
    out_type=jax.ShapeDtypeStruct((_SEQ, _D), jnp.float32),
    mesh=plsc.VectorSubcoreMesh(core_axis_name="c", subcore_axis_name="s"),
    scratch_types=[
        pltpu.VMEM((_NCH, _C), jnp.int32),
        pltpu.VMEM((_NBUF, _C, _D), jnp.float32),
        pltpu.VMEM((_PBUF, _C, _D), jnp.float32),
        pltpu.SemaphoreType.DMA((_NBUF,)),
        pltpu.SemaphoreType.DMA((_PBUF,)),
        pltpu.SemaphoreType.DMA((_NBUF,)),
    ],
)(_body)


def kernel(x, table):
    pe = jnp.asarray(_PE_NP)
    x3 = x.astype(jnp.int32).reshape(_NW, _NCH, _C)
    return _sc_kernel(x3, table, pe)


# speedup vs baseline: 1.0826x; 1.0826x over previous
import functools

import numpy as np
import jax
import jax.numpy as jnp
from jax import lax
from jax.experimental import pallas as pl
from jax.experimental.pallas import tpu as pltpu
from jax.experimental.pallas import tpu_sc as plsc

_SEQ = 8192
_D = 512
_LANES = 16
_NC = 2
_NS = 16
_NW = _NC * _NS
_BPW = _SEQ // _NW
_C = 32
_NCH = _BPW // _C
_NBUF = 4
_PBUF = 2


def _positional_encodings_np(seq_len, d):
    pos = np.arange(seq_len, dtype=np.float64)[:, None]
    hid = np.arange(d, dtype=np.float64)[None, :]
    angles = pos / np.power(10000.0, 2.0 * (np.floor(hid / 2.0)) / d)
    pe = np.array(angles)
    pe[:, 0::2] = np.sin(angles[:, 0::2])
    pe[:, 1::2] = np.cos(angles[:, 1::2])
    return pe.astype(np.float32)


_PE_NP = _positional_encodings_np(_SEQ, _D)


def _body(x3_hbm, table_hbm, pe_hbm, out_hbm, idx_v, rows_v, pe_v,
          sem_g, sem_p, sem_o):
    wid = lax.axis_index("s") * _NC + lax.axis_index("c")
    base = wid * _BPW
    pltpu.sync_copy(x3_hbm.at[wid], idx_v)

    def start_gather(ch):
        return pltpu.async_copy(
            table_hbm.at[idx_v.at[ch]], rows_v.at[ch % _NBUF],
            sem_g.at[ch % _NBUF])

    def start_pe(ch):
        return pltpu.async_copy(
            pe_hbm.at[pl.ds(base + ch * _C, _C)], pe_v.at[ch % _PBUF],
            sem_p.at[ch % _PBUF])

    g, p, o = {}, {}, {}
    for ch in range(_PBUF):
        g[ch] = start_gather(ch)
        p[ch] = start_pe(ch)

    for ch in range(_NCH):
        b = ch % _NBUF
        pb = ch % _PBUF
        nxt = ch + _PBUF
        if nxt < _NCH:
            if nxt - _NBUF >= 0:
                o[nxt - _NBUF].wait()
            g[nxt] = start_gather(nxt)
        g[ch].wait()
        p[ch].wait()

        @plsc.parallel_loop(0, _C, step=1, unroll=2)
        def _add(r):
            for j in range(_D // _LANES):
                off = j * _LANES
                plsc.addupdate(
                    rows_v.at[b, r, pl.ds(off, _LANES)],
                    pe_v[pb, r, pl.ds(off, _LANES)],
                )

        o[ch] = pltpu.async_copy(
            rows_v.at[b], out_hbm.at[pl.ds(base + ch * _C, _C)], sem_o.at[b])
        if nxt < _NCH:
            p[nxt] = start_pe(nxt)

    for ch in range(_NCH - min(_NBUF, _NCH), _NCH):
        o[ch].wait()


_sc_kernel = functools.partial(
    pl.kernel,
    out_type=jax.ShapeDtypeStruct((_SEQ, _D), jnp.float32),
    mesh=plsc.VectorSubcoreMesh(core_axis_name="c", subcore_axis_name="s"),
    scratch_types=[
        pltpu.VMEM((_NCH, _C), jnp.int32),
        pltpu.VMEM((_NBUF, _C, _D), jnp.float32),
        pltpu.VMEM((_PBUF, _C, _D), jnp.float32),
        pltpu.SemaphoreType.DMA((_NBUF,)),
        pltpu.SemaphoreType.DMA((_PBUF,)),
        pltpu.SemaphoreType.DMA((_NBUF,)),
    ],
)(_body)


def kernel(x, table):
    pe = jnp.asarray(_PE_NP)
    x3 = x.astype(jnp.int32).reshape(_NW, _NCH, _C)
    return _sc_kernel(x3, table, pe)


# speedup vs baseline: 1.1463x; 1.0589x over previous
import functools

import numpy as np
import jax
import jax.numpy as jnp
from jax import lax
from jax.experimental import pallas as pl
from jax.experimental.pallas import tpu as pltpu
from jax.experimental.pallas import tpu_sc as plsc

_SEQ = 8192
_D = 512
_LANES = 16
_NC = 2
_NS = 16
_NW = _NC * _NS
_BPW = _SEQ // _NW
_C = 32
_NCH = _BPW // _C
_NBUF = 4
_PBUF = 2


def _positional_encodings_np(seq_len, d):
    pos = np.arange(seq_len, dtype=np.float64)[:, None]
    hid = np.arange(d, dtype=np.float64)[None, :]
    angles = pos / np.power(10000.0, 2.0 * (np.floor(hid / 2.0)) / d)
    pe = np.array(angles)
    pe[:, 0::2] = np.sin(angles[:, 0::2])
    pe[:, 1::2] = np.cos(angles[:, 1::2])
    return pe.astype(np.float32)


_PE_NP = _positional_encodings_np(_SEQ, _D).reshape(-1)


def _body(x_hbm, table_hbm, pe_hbm, out_hbm, idx_v, rows_v, pe_v,
          sem_g, sem_p, sem_o):
    wid = lax.axis_index("s") * _NC + lax.axis_index("c")
    base = wid * _BPW
    pltpu.sync_copy(x_hbm.at[pl.ds(base, _BPW)], idx_v)

    def start_gather(ch):
        return pltpu.async_copy(
            table_hbm.at[idx_v.at[pl.ds(ch * _C, _C)]], rows_v.at[ch % _NBUF],
            sem_g.at[ch % _NBUF])

    def start_pe(ch):
        return pltpu.async_copy(
            pe_hbm.at[pl.ds((base + ch * _C) * _D, _C * _D)],
            pe_v.at[ch % _PBUF], sem_p.at[ch % _PBUF])

    g, p, o = {}, {}, {}
    for ch in range(_PBUF):
        g[ch] = start_gather(ch)
        p[ch] = start_pe(ch)

    for ch in range(_NCH):
        b = ch % _NBUF
        pb = ch % _PBUF
        nxt = ch + _PBUF
        if nxt < _NCH:
            if nxt - _NBUF >= 0:
                o[nxt - _NBUF].wait()
            g[nxt] = start_gather(nxt)
        g[ch].wait()
        p[ch].wait()

        @plsc.parallel_loop(0, _C, step=1, unroll=2)
        def _add(r):
            for j in range(_D // _LANES):
                off = j * _LANES
                plsc.addupdate(
                    rows_v.at[b, r, pl.ds(off, _LANES)],
                    pe_v[pb, pl.ds(r * _D + off, _LANES)],
                )

        o[ch] = pltpu.async_copy(
            rows_v.at[b], out_hbm.at[pl.ds(base + ch * _C, _C)], sem_o.at[b])
        if nxt < _NCH:
            p[nxt] = start_pe(nxt)

    for ch in range(_NCH - min(_NBUF, _NCH), _NCH):
        o[ch].wait()


_sc_kernel = functools.partial(
    pl.kernel,
    out_type=jax.ShapeDtypeStruct((_SEQ, _D), jnp.float32),
    mesh=plsc.VectorSubcoreMesh(core_axis_name="c", subcore_axis_name="s"),
    scratch_types=[
        pltpu.VMEM((_BPW,), jnp.int32),
        pltpu.VMEM((_NBUF, _C, _D), jnp.float32),
        pltpu.VMEM((_PBUF, _C * _D), jnp.float32),
        pltpu.SemaphoreType.DMA((_NBUF,)),
        pltpu.SemaphoreType.DMA((_PBUF,)),
        pltpu.SemaphoreType.DMA((_NBUF,)),
    ],
)(_body)


def kernel(x, table):
    pe = jnp.asarray(_PE_NP)
    return _sc_kernel(x.astype(jnp.int32), table, pe)


# speedup vs baseline: 1.2002x; 1.0470x over previous
import functools

import numpy as np
import jax
import jax.numpy as jnp
from jax import lax
from jax.experimental import pallas as pl
from jax.experimental.pallas import tpu as pltpu
from jax.experimental.pallas import tpu_sc as plsc

_SEQ = 8192
_D = 512
_K = _D // 2
_LANES = 16
_NC = 2
_NS = 16
_NW = _NC * _NS
_BPW = _SEQ // _NW
_C = 32
_NCH = _BPW // _C
_NBUF = 4
_PF = 2


def _pe_tables_np():
    om = 1.0 / np.power(10000.0, 2.0 * np.arange(_K, dtype=np.float64) / _D)
    w_ = np.arange(_NW, dtype=np.float64)
    a_ = np.arange(16, dtype=np.float64)
    b_ = np.arange(16, dtype=np.float64)

    def inter(x, y):
        out = np.empty(x.shape[:-1] + (_D,), np.float64)
        out[..., 0::2] = x
        out[..., 1::2] = y
        return out

    ang_wa = (256.0 * w_[:, None, None] + 16.0 * a_[None, :, None]) * om
    sw, cw = np.sin(ang_wa), np.cos(ang_wa)
    wa = np.stack([inter(sw, cw), inter(cw, -sw)], axis=2).astype(np.float32)
    ang_b = b_[:, None] * om[None, :]
    sb, cb = np.sin(ang_b), np.cos(ang_b)
    bt = np.stack([inter(cb, cb), inter(sb, sb)], axis=1).astype(np.float32)
    return wa, bt


_WA_NP, _BT_NP = _pe_tables_np()


def _body(x_hbm, table_hbm, wa_hbm, bt_hbm, out_hbm,
          idx_v, wa_v, bt_v, rows_v, sem_g, sem_o):
    wid = lax.axis_index("s") * _NC + lax.axis_index("c")
    base = wid * _BPW
    pltpu.sync_copy(x_hbm.at[pl.ds(base, _BPW)], idx_v)
    pltpu.sync_copy(wa_hbm.at[wid], wa_v)
    pltpu.sync_copy(bt_hbm, bt_v)

    def start_gather(ch):
        return pltpu.async_copy(
            table_hbm.at[idx_v.at[pl.ds(ch * _C, _C)]], rows_v.at[ch % _NBUF],
            sem_g.at[ch % _NBUF])

    g, o = {}, {}
    for ch in range(_PF):
        g[ch] = start_gather(ch)

    for ch in range(_NCH):
        b = ch % _NBUF
        nxt = ch + _PF
        if nxt < _NCH:
            if nxt - _NBUF >= 0:
                o[nxt - _NBUF].wait()
            g[nxt] = start_gather(nxt)
        g[ch].wait()

        for a_off in range(2):
            a = 2 * ch + a_off

            def g_body(gg, _):
                goff = gg * _LANES
                av = wa_v[a, 0, pl.ds(goff, _LANES)]
                cv = wa_v[a, 1, pl.ds(goff, _LANES)]

                @plsc.parallel_loop(0, 16, step=1, unroll=4)
                def _rows(bb):
                    bv = bt_v[bb, 0, pl.ds(goff, _LANES)]
                    dv = bt_v[bb, 1, pl.ds(goff, _LANES)]
                    plsc.addupdate(
                        rows_v.at[b, a_off * 16 + bb, pl.ds(goff, _LANES)],
                        av * bv + cv * dv,
                    )

                return 0

            lax.fori_loop(0, _D // _LANES, g_body, 0)

        o[ch] = pltpu.async_copy(
            rows_v.at[b], out_hbm.at[pl.ds(base + ch * _C, _C)], sem_o.at[b])

    for ch in range(_NCH - min(_NBUF, _NCH), _NCH):
        o[ch].wait()


_sc_kernel = functools.partial(
    pl.kernel,
    out_type=jax.ShapeDtypeStruct((_SEQ, _D), jnp.float32),
    mesh=plsc.VectorSubcoreMesh(core_axis_name="c", subcore_axis_name="s"),
    scratch_types=[
        pltpu.VMEM((_BPW,), jnp.int32),
        pltpu.VMEM((16, 2, _D), jnp.float32),
        pltpu.VMEM((16, 2, _D), jnp.float32),
        pltpu.VMEM((_NBUF, _C, _D), jnp.float32),
        pltpu.SemaphoreType.DMA((_NBUF,)),
        pltpu.SemaphoreType.DMA((_NBUF,)),
    ],
)(_body)


def kernel(x, table):
    wa = jnp.asarray(_WA_NP)
    bt = jnp.asarray(_BT_NP)
    return _sc_kernel(x.astype(jnp.int32), table, wa, bt)


# speedup vs baseline: 1.3711x; 1.1424x over previous
import functools

import numpy as np
import jax
import jax.numpy as jnp
from jax import lax
from jax.experimental import pallas as pl
from jax.experimental.pallas import tpu as pltpu
from jax.experimental.pallas import tpu_sc as plsc

_SEQ = 8192
_D = 512
_K = _D // 2
_LANES = 16
_NC = 2
_NS = 16
_NW = _NC * _NS
_BPW = _SEQ // _NW
_C = 32
_NCH = _BPW // _C
_NBUF = 4
_PF = 2


def _pe_tables_np():
    om = 1.0 / np.power(10000.0, 2.0 * np.arange(_K, dtype=np.float64) / _D)
    w_ = np.arange(_NW, dtype=np.float64)
    a_ = np.arange(16, dtype=np.float64)
    b_ = np.arange(16, dtype=np.float64)

    def inter(x, y):
        out = np.empty(x.shape[:-1] + (_D,), np.float64)
        out[..., 0::2] = x
        out[..., 1::2] = y
        return out

    ang_wa = (256.0 * w_[:, None, None] + 16.0 * a_[None, :, None]) * om
    sw, cw = np.sin(ang_wa), np.cos(ang_wa)
    wa = np.stack([inter(sw, cw), inter(cw, -sw)], axis=2).astype(np.float32)
    ang_b = b_[:, None] * om[None, :]
    sb, cb = np.sin(ang_b), np.cos(ang_b)
    bt = np.stack([inter(cb, cb), inter(sb, sb)], axis=1).astype(np.float32)
    return wa, bt


_WA_NP, _BT_NP = _pe_tables_np()


def _body(x_hbm, table_hbm, wa_hbm, bt_hbm, out_hbm,
          idx_v, wa_v, bt_v, rows_v, sem_g, sem_o):
    wid = lax.axis_index("s") * _NC + lax.axis_index("c")
    base = wid * _BPW
    pltpu.sync_copy(x_hbm.at[pl.ds(base, _BPW)], idx_v)
    pltpu.sync_copy(wa_hbm.at[wid], wa_v)
    pltpu.sync_copy(bt_hbm, bt_v)

    def start_gather(ch):
        return pltpu.async_copy(
            table_hbm.at[idx_v.at[pl.ds(ch * _C, _C)]], rows_v.at[ch % _NBUF],
            sem_g.at[ch % _NBUF])

    g, o = {}, {}
    for ch in range(_PF):
        g[ch] = start_gather(ch)

    for ch in range(_NCH):
        b = ch % _NBUF
        nxt = ch + _PF
        if nxt < _NCH:
            if nxt - _NBUF >= 0:
                o[nxt - _NBUF].wait()
            g[nxt] = start_gather(nxt)
        g[ch].wait()

        def g_body(gg, _):
            goff = gg * _LANES
            bd = [(bt_v[bb, 0, pl.ds(goff, _LANES)],
                   bt_v[bb, 1, pl.ds(goff, _LANES)]) for bb in range(16)]
            for a_off in range(2):
                a = 2 * ch + a_off
                av = wa_v[a, 0, pl.ds(goff, _LANES)]
                cv = wa_v[a, 1, pl.ds(goff, _LANES)]
                for bb in range(16):
                    bv, dv = bd[bb]
                    plsc.addupdate(
                        rows_v.at[b, a_off * 16 + bb, pl.ds(goff, _LANES)],
                        av * bv + cv * dv,
                    )
            return 0

        lax.fori_loop(0, _D // _LANES, g_body, 0)

        o[ch] = pltpu.async_copy(
            rows_v.at[b], out_hbm.at[pl.ds(base + ch * _C, _C)], sem_o.at[b])

    for ch in range(_NCH - min(_NBUF, _NCH), _NCH):
        o[ch].wait()


_sc_kernel = functools.partial(
    pl.kernel,
    out_type=jax.ShapeDtypeStruct((_SEQ, _D), jnp.float32),
    mesh=plsc.VectorSubcoreMesh(core_axis_name="c", subcore_axis_name="s"),
    scratch_types=[
        pltpu.VMEM((_BPW,), jnp.int32),
        pltpu.VMEM((16, 2, _D), jnp.float32),
        pltpu.VMEM((16, 2, _D), jnp.float32),
        pltpu.VMEM((_NBUF, _C, _D), jnp.float32),
        pltpu.SemaphoreType.DMA((_NBUF,)),
        pltpu.SemaphoreType.DMA((_NBUF,)),
    ],
)(_body)


def kernel(x, table):
    wa = jnp.asarray(_WA_NP)
    bt = jnp.asarray(_BT_NP)
    return _sc_kernel(x.astype(jnp.int32), table, wa, bt)
